# baseline (speedup 1.0000x reference)
import functools

import numpy as np
import jax
import jax.numpy as jnp
from jax import lax
from jax.experimental import pallas as pl
from jax.experimental.pallas import tpu as pltpu
from jax.experimental.pallas import tpu_sc as plsc

N = 10000
NPAD = 10240
RADIUS = 0.125
K = 16
CHUNK = 32
CHUNK16 = CHUNK * K
NCHUNK = NPAD // CHUNK
BLK = 128
EPAD_CAP = 208000
NBLK_CAP = EPAD_CAP // BLK
NBLK_PAD = 1648
NWORK = 32
RBLK = 256
f32 = jnp.float32
i32 = jnp.int32

_mesh = plsc.VectorSubcoreMesh(core_axis_name="c", subcore_axis_name="s")


def _hypot16(ax, ay):
    m = jnp.maximum(ax, ay)
    msafe = jnp.maximum(m, 1e-12)
    t = jnp.minimum(ax, ay) / msafe
    y = 1.0 + t * t
    r = (0.14632082 * y - 0.72323499) * y + 1.57186441
    r = r * (1.5 - 0.5 * y * r * r)
    r = r * (1.5 - 0.5 * y * r * r)
    return m * (y * r)


def _atan16(t):
    t2 = t * t
    p = jnp.full((16,), -0.0040540580, f32)
    for c in (0.0218612288, -0.0559098861, 0.0964200441, -0.1390853351,
              0.1994653599, -0.3332985605, 0.9999993329):
        p = p * t2 + c
    return t * p


def _meta_body(posx, posy, ein, blkb, mout,
               erowA, erowB, svA, svB, dvA, dvB,
               psxA, psyA, pdxA, pdyA, psxB, psyB, pdxB, pdyB,
               orow, blkv,
               sA0, sA1, sA2, sA3, sB0, sB1, sB2, sB3):
    wid = lax.axis_index("s") * 2 + lax.axis_index("c")
    pltpu.sync_copy(blkb, blkv)
    nbtot = blkv[pl.ds(NCHUNK, 16)][0]
    cntw = (nbtot - wid + (NWORK - 1)) // NWORK

    bufs = ((erowA, svA, dvA, psxA, psyA, pdxA, pdyA, sA0, sA1, sA2, sA3),
            (erowB, svB, dvB, psxB, psyB, pdxB, pdyB, sB0, sB1, sB2, sB3))

    def prefetch(t, p):
        erow, sv, dv, psx, psy, pdx, pdy, s0, s1, s2, s3 = bufs[p]
        b = wid + t * NWORK
        pltpu.sync_copy(ein.at[b], erow)
        for g in range(BLK // 16):
            s = g * 16
            sv[pl.ds(s, 16)] = erow[pl.ds(s, 16)].astype(i32)
            dv[pl.ds(s, 16)] = erow[pl.ds(BLK + s, 16)].astype(i32)
        pltpu.async_copy(posx.at[sv], psx, s0)
        pltpu.async_copy(posy.at[sv], psy, s1)
        pltpu.async_copy(posx.at[dv], pdx, s2)
        pltpu.async_copy(posy.at[dv], pdy, s3)

    def process(t, p):
        erow, sv, dv, psx, psy, pdx, pdy, s0, s1, s2, s3 = bufs[p]
        b = wid + t * NWORK
        pltpu.make_async_copy(posx.at[sv], psx, s0).wait()
        pltpu.make_async_copy(posy.at[sv], psy, s1).wait()
        pltpu.make_async_copy(posx.at[dv], pdx, s2).wait()
        pltpu.make_async_copy(posy.at[dv], pdy, s3).wait()
        n0f = erow[pl.ds(3 * BLK, 16)]

        def grp(g, c4):
            s = g * 16
            sx = psx[pl.ds(s, 16)]
            sy = psy[pl.ds(s, 16)]
            dx = pdx[pl.ds(s, 16)]
            dy = pdy[pl.ds(s, 16)]
            dlf = erow[pl.ds(BLK + s, 16)]
            vl = erow[pl.ds(2 * BLK + s, 16)]
            rx = (sx - dx) * (1.0 / RADIUS)
            ry = (sy - dy) * (1.0 / RADIUS)
            sq = rx * rx + ry * ry
            om = 1.0 - sq
            win = jnp.clip(om * om * om, 0.0, 1.0)
            ax = jnp.abs(rx)
            ay = jnp.abs(ry)
            nxy = _hypot16(ax, ay)
            condx = ax >= ay
            xs = jnp.where(ax > 1e-8, rx, 1.0)
            ys = jnp.where(ay > 1e-8, ry, 1.0)
            FOUR_PI = 1.2732395447351628
            a1 = _atan16(jnp.clip(ry / xs, -1.0, 1.0))
            u1 = jnp.sign(rx) * nxy
            v1 = u1 * FOUR_PI * a1
            a2 = _atan16(jnp.clip(rx / ys, -1.0, 1.0))
            v2 = jnp.sign(ry) * nxy
            u2 = v2 * FOUR_PI * a2
            u = jnp.where(condx, u1, u2)
            v = jnp.where(condx, v1, v2)
            tiny = sq < 1e-12
            u = jnp.where(tiny, 0.0, u)
            v = jnp.where(tiny, 0.0, v)
            gx = jnp.clip((u + 1.0) * 1.5, 0.0, 3.0)
            gy = jnp.clip((v + 1.0) * 1.5, 0.0, 3.0)
            x0i = jnp.minimum(gx.astype(i32), 2)
            y0i = jnp.minimum(gy.astype(i32), 2)
            wx1 = gx - x0i.astype(f32)
            wy1 = gy - y0i.astype(f32)
            wx0 = 1.0 - wx1
            wy0 = 1.0 - wy1
            wv_ = win * vl
            cellf = (x0i * 4 + y0i).astype(f32)
            orow[pl.ds(s, 16)] = (dlf - n0f) * float(K) + cellf
            orow[pl.ds(BLK + s, 16)] = wx0 * wy0 * wv_
            orow[pl.ds(2 * BLK + s, 16)] = wx0 * wy1 * wv_
            orow[pl.ds(3 * BLK + s, 16)] = wx1 * wy0 * wv_
            orow[pl.ds(4 * BLK + s, 16)] = wx1 * wy1 * wv_
            orow[pl.ds(5 * BLK + s, 16)] = erow[pl.ds(s, 16)]
            return c4

        lax.fori_loop(0, BLK // 16, grp, 0)
        pltpu.sync_copy(orow, mout.at[b])

    @pl.when(cntw > 0)
    def _():
        prefetch(0, 0)

    def pair_body(pair, carry):
        tA = 2 * pair
        tB = tA + 1

        @pl.when(tB < cntw)
        def _():
            prefetch(tB, 1)

        @pl.when(tA < cntw)
        def _():
            process(tA, 0)

        @pl.when(tB + 1 < cntw)
        def _():
            prefetch(tB + 1, 0)

        @pl.when(tB < cntw)
        def _():
            process(tB, 1)
        return carry

    lax.fori_loop(0, (cntw + 1) // 2, pair_body, 0)


_sc_meta = pl.kernel(
    _meta_body,
    out_type=jax.ShapeDtypeStruct((NBLK_CAP, 6 * BLK), f32),
    mesh=_mesh,
    scratch_types=[
        pltpu.VMEM((4 * BLK,), f32),
        pltpu.VMEM((4 * BLK,), f32),
        pltpu.VMEM((BLK,), i32),
        pltpu.VMEM((BLK,), i32),
        pltpu.VMEM((BLK,), i32),
        pltpu.VMEM((BLK,), i32),
        pltpu.VMEM((BLK,), f32),
        pltpu.VMEM((BLK,), f32),
        pltpu.VMEM((BLK,), f32),
        pltpu.VMEM((BLK,), f32),
        pltpu.VMEM((BLK,), f32),
        pltpu.VMEM((BLK,), f32),
        pltpu.VMEM((BLK,), f32),
        pltpu.VMEM((BLK,), f32),
        pltpu.VMEM((6 * BLK,), f32),
        pltpu.VMEM((NCHUNK + 16,), i32),
        pltpu.SemaphoreType.DMA,
        pltpu.SemaphoreType.DMA,
        pltpu.SemaphoreType.DMA,
        pltpu.SemaphoreType.DMA,
        pltpu.SemaphoreType.DMA,
        pltpu.SemaphoreType.DMA,
        pltpu.SemaphoreType.DMA,
        pltpu.SemaphoreType.DMA,
    ],
)


def _make_build_h(ci, relu):
    qn = ci // 16
    epr = 128 // ci if ci < 128 else 1
    rpg = (16 * ci + 127) // 128
    nrow = BLK * ci // 128 if ci < 128 else BLK

    def body(feats, mout, wctrl, blkb, h_out,
             mrowA, mrowB, svA, svB, fbufA, fbufB, hloc, ctrlv, blkv,
             semA, semB):
        wid = lax.axis_index("s") * 2 + lax.axis_index("c")
        pltpu.sync_copy(wctrl, ctrlv)
        pltpu.sync_copy(blkb, blkv)
        cv = ctrlv[pl.ds(wid, 16)]
        c_lo = cv[0]
        c_hi = cv[1]

        iota16 = jnp.arange(16, dtype=i32)
        bufs = ((mrowA, svA, fbufA, semA), (mrowB, svB, fbufB, semB))

        def issue_gather(p):
            mrow, sv, fbuf, sem = bufs[p]
            if ci < 128:
                for r in range(nrow):
                    pltpu.async_copy(feats.at[sv.at[r]], fbuf.at[r], sem)
            else:
                pltpu.async_copy(feats.at[sv], fbuf, sem)

        def drain_gather(p):
            mrow, sv, fbuf, sem = bufs[p]
            if ci < 128:
                for r in range(nrow):
                    pltpu.make_async_copy(feats.at[sv.at[r]], fbuf.at[r],
                                          sem).wait()
            else:
                pltpu.make_async_copy(feats.at[sv], fbuf, sem).wait()

        def prefetch(b, p):
            mrow, sv, fbuf, sem = bufs[p]
            pltpu.sync_copy(mout.at[b], mrow)
            if ci < 128:
                for g in range(BLK // 16):
                    s = g * 16
                    bvi = (mrow[pl.ds(5 * BLK + s, 16)] * 128.0).astype(i32)
                    for l in range(16):
                        bsc = bvi[l]
                        row = g * rpg + l // epr
                        c0 = ci * (l % epr)
                        for q in range(qn):
                            sv[row, pl.ds(c0 + q * 16, 16)] = (
                                bsc + (iota16 + q * 16))
            else:
                for g in range(BLK // 16):
                    s = g * 16
                    sv[pl.ds(s, 16)] = mrow[pl.ds(5 * BLK + s, 16)].astype(i32)
            issue_gather(p)

        def process(p):
            mrow, sv, fbuf, sem = bufs[p]
            drain_gather(p)

            def grp(g, c4):
                s = g * 16
                rv = mrow[pl.ds(s, 16)].astype(i32)
                w0v = mrow[pl.ds(BLK + s, 16)]
                w1v = mrow[pl.ds(2 * BLK + s, 16)]
                w2v = mrow[pl.ds(3 * BLK + s, 16)]
                w3v = mrow[pl.ds(4 * BLK + s, 16)]
                for l in range(16):
                    r = rv[l]
                    for q in range(qn):
                        sl = pl.ds(q * 16, 16)
                        if ci < 128:
                            f = fbuf[g * rpg + l // epr,
                                     pl.ds(ci * (l % epr) + q * 16, 16)]
                        else:
                            f = fbuf[s + l, sl]
                        if relu:
                            f = jnp.maximum(f, 0.0)
                        plsc.addupdate(hloc.at[r, sl], w0v[l] * f)
                        plsc.addupdate(hloc.at[r + 1, sl], w1v[l] * f)
                        plsc.addupdate(hloc.at[r + 4, sl], w2v[l] * f)
                        plsc.addupdate(hloc.at[r + 5, sl], w3v[l] * f)
                return c4
            lax.fori_loop(0, BLK // 16, grp, 0)

        def chunk_body(j, carry):
            def zrow(r, c2):
                for q in range(qn):
                    hloc[r, pl.ds(q * 16, 16)] = jnp.zeros((16,), f32)
                return c2
            lax.fori_loop(0, CHUNK16, zrow, 0)
            bv = blkv[pl.ds(j, 16)]
            b0 = bv[0]
            nb = bv[1] - b0

            @pl.when(nb > 0)
            def _():
                prefetch(b0, 0)

            def pair_body(pair, c3):
                tA = 2 * pair
                tB = tA + 1

                @pl.when(tB < nb)
                def _():
                    prefetch(b0 + tB, 1)

                @pl.when(tA < nb)
                def _():
                    process(0)

                @pl.when(tB + 1 < nb)
                def _():
                    prefetch(b0 + tB + 1, 0)

                @pl.when(tB < nb)
                def _():
                    process(1)
                return c3

            lax.fori_loop(0, (nb + 1) // 2, pair_body, 0)
            pltpu.sync_copy(hloc, h_out.at[pl.ds(j * CHUNK16, CHUNK16)])
            return carry
        lax.fori_loop(c_lo, c_hi, chunk_body, 0)

    if ci < 128:
        sv_t = pltpu.VMEM((nrow, 128), i32)
        fb_t = pltpu.VMEM((nrow, 128), f32)
    else:
        sv_t = pltpu.VMEM((BLK,), i32)
        fb_t = pltpu.VMEM((BLK, 128), f32)
    return pl.kernel(
        body,
        out_type=jax.ShapeDtypeStruct((NPAD * K, ci), f32),
        mesh=_mesh,
        scratch_types=[
            pltpu.VMEM((6 * BLK,), f32),
            pltpu.VMEM((6 * BLK,), f32),
            sv_t,
            sv_t,
            fb_t,
            fb_t,
            pltpu.VMEM((CHUNK16, ci), f32),
            pltpu.VMEM((48,), i32),
            pltpu.VMEM((NCHUNK + 16,), i32),
            pltpu.SemaphoreType.DMA,
            pltpu.SemaphoreType.DMA,
        ],
    )


_build_h16 = _make_build_h(16, False)
_build_h64 = _make_build_h(64, True)
_build_h128 = _make_build_h(128, True)


def _tc_bn(vel128, g128, b128):
    def body(v_ref, g_ref, b_ref, o_ref):
        v = v_ref[...]
        s1 = jnp.sum(v, axis=0, keepdims=True) * (1.0 / N)
        s2 = jnp.sum(v * v, axis=0, keepdims=True) * (1.0 / N)
        var = s2 - s1 * s1
        o_ref[...] = (v - s1) * lax.rsqrt(var + 1e-5) * g_ref[...] + b_ref[...]
    return pl.pallas_call(
        body, out_shape=jax.ShapeDtypeStruct((NPAD, 128), f32),
    )(vel128, g128, b128)


def _tc_layer(h2, feats, A, D, bias, res=None, WG=None, act=None,
              relu_feats=True):
    kci = A.shape[0]
    co = A.shape[1]
    ci = feats.shape[1]
    has_res = res is not None
    has_g = WG is not None

    def body(h_ref, f_ref, A_ref, D_ref, b_ref, *rest):
        rest = list(rest)
        res_ref = rest.pop(0) if has_res else None
        wg_ref = rest.pop(0) if has_g else None
        o_ref = rest.pop(0)
        g_ref = rest.pop(0) if has_g else None
        f = f_ref[...]
        if relu_feats:
            f = jnp.maximum(f, 0.0)
        x = (jnp.dot(h_ref[...], A_ref[...], preferred_element_type=f32)
             + jnp.dot(f, D_ref[...], preferred_element_type=f32)
             + b_ref[...])
        if has_res:
            x = x + res_ref[...]
        if act == "tanh":
            x = jnp.tanh(x) * 0.8 + 1.0
        o_ref[...] = x
        if has_g:
            g_ref[...] = jnp.dot(jnp.maximum(x, 0.0), wg_ref[...],
                                 preferred_element_type=f32)

    in_arrays = [h2, feats, A, D, bias]
    in_specs = [
        pl.BlockSpec((RBLK, kci), lambda i: (i, 0)),
        pl.BlockSpec((RBLK, ci), lambda i: (i, 0)),
        pl.BlockSpec((kci, co), lambda i: (0, 0)),
        pl.BlockSpec((ci, co), lambda i: (0, 0)),
        pl.BlockSpec((1, co), lambda i: (0, 0)),
    ]
    if has_res:
        in_arrays.append(res)
        in_specs.append(pl.BlockSpec((RBLK, co), lambda i: (i, 0)))
    if has_g:
        cg = WG.shape[1]
        in_arrays.append(WG)
        in_specs.append(pl.BlockSpec((co, cg), lambda i: (0, 0)))
    out_shape = [jax.ShapeDtypeStruct((NPAD, co), f32)]
    out_specs = [pl.BlockSpec((RBLK, co), lambda i: (i, 0))]
    if has_g:
        out_shape.append(jax.ShapeDtypeStruct((NPAD, cg), f32))
        out_specs.append(pl.BlockSpec((RBLK, cg), lambda i: (i, 0)))
    outs = pl.pallas_call(
        body, grid=(NPAD // RBLK,), in_specs=in_specs,
        out_specs=out_specs, out_shape=out_shape,
    )(*in_arrays)
    return outs


_WDIAG = np.zeros((256, 128), np.float32)
for _k in range(16):
    _WDIAG[_k * 16 + _k, 0] = 1.0


def kernel(pos, vel, edge_src, edge_dst, bn_gamma, bn_beta,
           conv0_W, conv0_b, dense0_W, dense0_b,
           conv1_W, conv1_b, dense1_W, dense1_b,
           conv2_W, conv2_b, dense2_W, dense2_b,
           conv3_W, conv3_b, dense3_W, dense3_b,
           conv4_W, conv4_b, dense4_W, dense4_b):
    E = edge_src.shape[0]
    src = edge_src.astype(i32)
    dst = edge_dst.astype(i32)

    chunk_of_edge = dst // CHUNK
    cnt = jnp.bincount(chunk_of_edge, length=NCHUNK).astype(i32)
    capblk = (cnt + (BLK - 1)) // BLK
    blkb = jnp.concatenate([jnp.zeros((1,), i32),
                            jnp.cumsum(capblk).astype(i32)])
    estart = jnp.concatenate([jnp.zeros((1,), i32),
                              jnp.cumsum(cnt).astype(i32)])
    slot = (blkb[chunk_of_edge] * BLK
            + (jnp.arange(E, dtype=i32) - estart[chunk_of_edge]))
    blk_ids = jnp.arange(NBLK_CAP, dtype=i32)
    chunk_of_blk = jnp.clip(
        jnp.searchsorted(blkb, blk_ids, side="right").astype(i32) - 1,
        0, NCHUNK - 1)
    n0blk = (chunk_of_blk * CHUNK).astype(i32)
    n0slot = jnp.repeat(n0blk, BLK)
    srcf = jnp.zeros((EPAD_CAP,), f32).at[slot].set(src.astype(f32))
    dstf = n0slot.astype(f32).at[slot].set(dst.astype(f32))
    validf = jnp.zeros((EPAD_CAP,), f32).at[slot].set(1.0)
    ein = jnp.concatenate([
        srcf.reshape(NBLK_CAP, BLK),
        dstf.reshape(NBLK_CAP, BLK),
        validf.reshape(NBLK_CAP, BLK),
        jnp.broadcast_to(n0blk.astype(f32)[:, None], (NBLK_CAP, 16)),
        jnp.zeros((NBLK_CAP, BLK - 16), f32),
    ], axis=1)
    nbtot = blkb[NCHUNK]
    targets = (jnp.arange(NWORK + 1, dtype=i32) * nbtot) // NWORK
    wctrl = jnp.searchsorted(blkb, targets, side="left").astype(i32)
    wctrl = wctrl.at[NWORK].set(NCHUNK)
    wctrl_pad = jnp.zeros((48,), i32).at[:NWORK + 1].set(wctrl)
    blkb_pad = jnp.zeros((NCHUNK + 16,), i32).at[:NCHUNK + 1].set(blkb)

    posx = jnp.zeros((NPAD,), f32).at[:N].set(pos[:, 0])
    posy = jnp.zeros((NPAD,), f32).at[:N].set(pos[:, 1])
    mout = _sc_meta(posx, posy, ein, blkb_pad)

    vel128 = jnp.zeros((NPAD, 128), f32).at[:N, :2].set(vel)
    g128 = jnp.ones((128,), f32).at[:3].set(bn_gamma).reshape(1, 128)
    b128 = jnp.zeros((128,), f32).at[:3].set(bn_beta).reshape(1, 128)
    fl128 = _tc_bn(vel128, g128, b128)

    h0 = _build_h16(fl128.reshape(NPAD * 128), mout, wctrl_pad, blkb_pad)
    Wf0 = conv0_W.reshape(K, 3, 32)
    Wf0p = jnp.zeros((K, 16, 32), f32).at[:, :3].set(Wf0).reshape(256, 32)
    A0 = jnp.zeros((256, 128), f32).at[:, :32].set(Wf0p)
    D0 = jnp.zeros((128, 128), f32).at[:3, 32:64].set(dense0_W)
    bias0 = jnp.zeros((128,), f32).at[:32].set(conv0_b).at[32:64].set(
        dense0_b).reshape(1, 128)
    (x0,) = _tc_layer(h0.reshape(NPAD, 256), fl128, A0, D0, bias0,
                      relu_feats=False)

    h1 = _build_h64(x0.reshape(NPAD * 128), mout, wctrl_pad, blkb_pad)
    A1 = jnp.zeros((1024, 128), f32).at[:, :64].set(
        conv1_W.reshape(1024, 64))
    D1 = jnp.zeros((128, 128), f32).at[:64, :64].set(dense1_W)
    bias1 = jnp.zeros((128,), f32).at[:64].set(conv1_b + dense1_b).reshape(1, 128)
    (x1,) = _tc_layer(h1.reshape(NPAD, 1024), x0, A1, D1, bias1,
                      res=x0, relu_feats=True)

    h2 = _build_h64(x1.reshape(NPAD * 128), mout, wctrl_pad, blkb_pad)
    A2 = conv2_W.reshape(K, 64, 128).reshape(1024, 128)
    D2 = jnp.zeros((128, 128), f32).at[:64, :].set(dense2_W)
    bias2 = (conv2_b + dense2_b).reshape(1, 128)
    (x2,) = _tc_layer(h2.reshape(NPAD, 1024), x1, A2, D2, bias2,
                      relu_feats=True)

    h3 = _build_h128(x2, mout, wctrl_pad, blkb_pad)
    A3 = conv3_W.reshape(K, 128, 256).reshape(2048, 256)
    bias3 = (conv3_b + dense3_b).reshape(1, 256)
    WG = jnp.zeros((256, 128), f32).at[:, :K].set(
        conv4_W.reshape(K, 256).T)
    x3, G4 = _tc_layer(h3.reshape(NPAD, 2048), x2, A3, dense3_W, bias3,
                       WG=WG, relu_feats=True)

    h4 = _build_h16(G4.reshape(NPAD * 128), mout, wctrl_pad, blkb_pad)
    Wdiag = jnp.asarray(_WDIAG)
    D4 = jnp.zeros((256, 128), f32).at[:, :1].set(dense4_W)
    bias4 = jnp.zeros((1, 128), f32).at[0, 0].set(conv4_b[0] + dense4_b[0])
    (x4,) = _tc_layer(h4.reshape(NPAD, 256), x3, Wdiag, D4, bias4,
                      act="tanh", relu_feats=True)
    return x4[:N, :1]


# speedup vs baseline: 1.9867x; 1.9867x over previous
import functools

import numpy as np
import jax
import jax.numpy as jnp
from jax import lax
from jax.experimental import pallas as pl
from jax.experimental.pallas import tpu as pltpu
from jax.experimental.pallas import tpu_sc as plsc

N = 10000
NPAD = 10240
RADIUS = 0.125
K = 16
CHUNK = 32
CHUNK16 = CHUNK * K
NCHUNK = NPAD // CHUNK
BLK = 128
EPAD_CAP = 208000
NBLK_CAP = EPAD_CAP // BLK
NBLK_PAD = 1648
NWORK = 32
RBLK = 256
f32 = jnp.float32
i32 = jnp.int32

_mesh = plsc.VectorSubcoreMesh(core_axis_name="c", subcore_axis_name="s")


def _hypot16(ax, ay):
    m = jnp.maximum(ax, ay)
    msafe = jnp.maximum(m, 1e-12)
    t = jnp.minimum(ax, ay) / msafe
    y = 1.0 + t * t
    r = (0.14632082 * y - 0.72323499) * y + 1.57186441
    r = r * (1.5 - 0.5 * y * r * r)
    r = r * (1.5 - 0.5 * y * r * r)
    return m * (y * r)


def _atan16(t):
    t2 = t * t
    p = jnp.full((16,), -0.0040540580, f32)
    for c in (0.0218612288, -0.0559098861, 0.0964200441, -0.1390853351,
              0.1994653599, -0.3332985605, 0.9999993329):
        p = p * t2 + c
    return t * p


def _meta_body(posx, posy, ein, blkb, mout,
               erowA, erowB, svA, svB, dvA, dvB,
               psxA, psyA, pdxA, pdyA, psxB, psyB, pdxB, pdyB,
               orow, blkv,
               sA0, sA1, sA2, sA3, sB0, sB1, sB2, sB3):
    wid = lax.axis_index("s") * 2 + lax.axis_index("c")
    pltpu.sync_copy(blkb, blkv)
    nbtot = blkv[pl.ds(NCHUNK, 16)][0]
    cntw = (nbtot - wid + (NWORK - 1)) // NWORK

    bufs = ((erowA, svA, dvA, psxA, psyA, pdxA, pdyA, sA0, sA1, sA2, sA3),
            (erowB, svB, dvB, psxB, psyB, pdxB, pdyB, sB0, sB1, sB2, sB3))

    def prefetch(t, p):
        erow, sv, dv, psx, psy, pdx, pdy, s0, s1, s2, s3 = bufs[p]
        b = wid + t * NWORK
        pltpu.sync_copy(ein.at[b], erow)
        for g in range(BLK // 16):
            s = g * 16
            sv[pl.ds(s, 16)] = erow[pl.ds(s, 16)].astype(i32)
            dv[pl.ds(s, 16)] = erow[pl.ds(BLK + s, 16)].astype(i32)
        pltpu.async_copy(posx.at[sv], psx, s0)
        pltpu.async_copy(posy.at[sv], psy, s1)
        pltpu.async_copy(posx.at[dv], pdx, s2)
        pltpu.async_copy(posy.at[dv], pdy, s3)

    def process(t, p):
        erow, sv, dv, psx, psy, pdx, pdy, s0, s1, s2, s3 = bufs[p]
        b = wid + t * NWORK
        pltpu.make_async_copy(posx.at[sv], psx, s0).wait()
        pltpu.make_async_copy(posy.at[sv], psy, s1).wait()
        pltpu.make_async_copy(posx.at[dv], pdx, s2).wait()
        pltpu.make_async_copy(posy.at[dv], pdy, s3).wait()
        n0f = erow[pl.ds(3 * BLK, 16)]

        def grp(g, c4):
            s = g * 16
            sx = psx[pl.ds(s, 16)]
            sy = psy[pl.ds(s, 16)]
            dx = pdx[pl.ds(s, 16)]
            dy = pdy[pl.ds(s, 16)]
            dlf = erow[pl.ds(BLK + s, 16)]
            vl = erow[pl.ds(2 * BLK + s, 16)]
            rx = (sx - dx) * (1.0 / RADIUS)
            ry = (sy - dy) * (1.0 / RADIUS)
            sq = rx * rx + ry * ry
            om = 1.0 - sq
            win = jnp.clip(om * om * om, 0.0, 1.0)
            ax = jnp.abs(rx)
            ay = jnp.abs(ry)
            nxy = _hypot16(ax, ay)
            condx = ax >= ay
            xs = jnp.where(ax > 1e-8, rx, 1.0)
            ys = jnp.where(ay > 1e-8, ry, 1.0)
            FOUR_PI = 1.2732395447351628
            a1 = _atan16(jnp.clip(ry / xs, -1.0, 1.0))
            u1 = jnp.sign(rx) * nxy
            v1 = u1 * FOUR_PI * a1
            a2 = _atan16(jnp.clip(rx / ys, -1.0, 1.0))
            v2 = jnp.sign(ry) * nxy
            u2 = v2 * FOUR_PI * a2
            u = jnp.where(condx, u1, u2)
            v = jnp.where(condx, v1, v2)
            tiny = sq < 1e-12
            u = jnp.where(tiny, 0.0, u)
            v = jnp.where(tiny, 0.0, v)
            gx = jnp.clip((u + 1.0) * 1.5, 0.0, 3.0)
            gy = jnp.clip((v + 1.0) * 1.5, 0.0, 3.0)
            x0i = jnp.minimum(gx.astype(i32), 2)
            y0i = jnp.minimum(gy.astype(i32), 2)
            wx1 = gx - x0i.astype(f32)
            wy1 = gy - y0i.astype(f32)
            wx0 = 1.0 - wx1
            wy0 = 1.0 - wy1
            wv_ = win * vl
            cellf = (x0i * 4 + y0i).astype(f32)
            orow[pl.ds(s, 16)] = (dlf - n0f) * float(K) + cellf
            orow[pl.ds(BLK + s, 16)] = wx0 * wy0 * wv_
            orow[pl.ds(2 * BLK + s, 16)] = wx0 * wy1 * wv_
            orow[pl.ds(3 * BLK + s, 16)] = wx1 * wy0 * wv_
            orow[pl.ds(4 * BLK + s, 16)] = wx1 * wy1 * wv_
            orow[pl.ds(5 * BLK + s, 16)] = erow[pl.ds(s, 16)]
            return c4

        lax.fori_loop(0, BLK // 16, grp, 0)
        pltpu.sync_copy(orow, mout.at[b])

    @pl.when(cntw > 0)
    def _():
        prefetch(0, 0)

    def pair_body(pair, carry):
        tA = 2 * pair
        tB = tA + 1

        @pl.when(tB < cntw)
        def _():
            prefetch(tB, 1)

        @pl.when(tA < cntw)
        def _():
            process(tA, 0)

        @pl.when(tB + 1 < cntw)
        def _():
            prefetch(tB + 1, 0)

        @pl.when(tB < cntw)
        def _():
            process(tB, 1)
        return carry

    lax.fori_loop(0, (cntw + 1) // 2, pair_body, 0)


_sc_meta = pl.kernel(
    _meta_body,
    out_type=jax.ShapeDtypeStruct((NBLK_CAP, 6 * BLK), f32),
    mesh=_mesh,
    scratch_types=[
        pltpu.VMEM((4 * BLK,), f32),
        pltpu.VMEM((4 * BLK,), f32),
        pltpu.VMEM((BLK,), i32),
        pltpu.VMEM((BLK,), i32),
        pltpu.VMEM((BLK,), i32),
        pltpu.VMEM((BLK,), i32),
        pltpu.VMEM((BLK,), f32),
        pltpu.VMEM((BLK,), f32),
        pltpu.VMEM((BLK,), f32),
        pltpu.VMEM((BLK,), f32),
        pltpu.VMEM((BLK,), f32),
        pltpu.VMEM((BLK,), f32),
        pltpu.VMEM((BLK,), f32),
        pltpu.VMEM((BLK,), f32),
        pltpu.VMEM((6 * BLK,), f32),
        pltpu.VMEM((NCHUNK + 16,), i32),
        pltpu.SemaphoreType.DMA,
        pltpu.SemaphoreType.DMA,
        pltpu.SemaphoreType.DMA,
        pltpu.SemaphoreType.DMA,
        pltpu.SemaphoreType.DMA,
        pltpu.SemaphoreType.DMA,
        pltpu.SemaphoreType.DMA,
        pltpu.SemaphoreType.DMA,
    ],
)


def _make_build_h(ci, relu):
    qn = ci // 16
    epr = 128 // ci if ci < 128 else 1
    rpg = (16 * ci + 127) // 128
    nrow = BLK * ci // 128 if ci < 128 else BLK

    def body(feats, mout, wctrl, blkb, h_out,
             mrowA, mrowB, svA, svB, fbufA, fbufB, hloc, ctrlv, blkv,
             semA, semB):
        wid = lax.axis_index("s") * 2 + lax.axis_index("c")
        pltpu.sync_copy(wctrl, ctrlv)
        pltpu.sync_copy(blkb, blkv)
        cv = ctrlv[pl.ds(wid, 16)]
        c_lo = cv[0]
        c_hi = cv[1]

        iota16 = jnp.arange(16, dtype=i32)
        bufs = ((mrowA, svA, fbufA, semA), (mrowB, svB, fbufB, semB))

        def issue_gather(p):
            mrow, sv, fbuf, sem = bufs[p]
            pltpu.async_copy(feats.at[sv], fbuf, sem)

        def drain_gather(p):
            mrow, sv, fbuf, sem = bufs[p]
            pltpu.make_async_copy(feats.at[sv], fbuf, sem).wait()

        def prefetch(b, p):
            mrow, sv, fbuf, sem = bufs[p]
            pltpu.sync_copy(mout.at[b], mrow)
            for g in range(BLK // 16):
                s = g * 16
                sv[pl.ds(s, 16)] = mrow[pl.ds(5 * BLK + s, 16)].astype(i32)
            issue_gather(p)

        def process(p):
            mrow, sv, fbuf, sem = bufs[p]
            drain_gather(p)

            def grp(g, c4):
                s = g * 16
                rv = mrow[pl.ds(s, 16)].astype(i32)
                w0v = mrow[pl.ds(BLK + s, 16)]
                w1v = mrow[pl.ds(2 * BLK + s, 16)]
                w2v = mrow[pl.ds(3 * BLK + s, 16)]
                w3v = mrow[pl.ds(4 * BLK + s, 16)]
                for l in range(16):
                    r = rv[l]
                    for q in range(qn):
                        sl = pl.ds(q * 16, 16)
                        f = fbuf[s + l, sl]
                        if relu:
                            f = jnp.maximum(f, 0.0)
                        plsc.addupdate(hloc.at[r, sl], w0v[l] * f)
                        plsc.addupdate(hloc.at[r + 1, sl], w1v[l] * f)
                        plsc.addupdate(hloc.at[r + 4, sl], w2v[l] * f)
                        plsc.addupdate(hloc.at[r + 5, sl], w3v[l] * f)
                return c4
            lax.fori_loop(0, BLK // 16, grp, 0)

        def chunk_body(j, carry):
            def zrow(r, c2):
                for q in range(qn):
                    hloc[r, pl.ds(q * 16, 16)] = jnp.zeros((16,), f32)
                return c2
            lax.fori_loop(0, CHUNK16, zrow, 0)
            bv = blkv[pl.ds(j, 16)]
            b0 = bv[0]
            nb = bv[1] - b0

            @pl.when(nb > 0)
            def _():
                prefetch(b0, 0)

            def pair_body(pair, c3):
                tA = 2 * pair
                tB = tA + 1

                @pl.when(tB < nb)
                def _():
                    prefetch(b0 + tB, 1)

                @pl.when(tA < nb)
                def _():
                    process(0)

                @pl.when(tB + 1 < nb)
                def _():
                    prefetch(b0 + tB + 1, 0)

                @pl.when(tB < nb)
                def _():
                    process(1)
                return c3

            lax.fori_loop(0, (nb + 1) // 2, pair_body, 0)
            pltpu.sync_copy(hloc, h_out.at[pl.ds(j * CHUNK16, CHUNK16)])
            return carry
        lax.fori_loop(c_lo, c_hi, chunk_body, 0)

    sv_t = pltpu.VMEM((BLK,), i32)
    fb_t = pltpu.VMEM((BLK, 128), f32)
    return pl.kernel(
        body,
        out_type=jax.ShapeDtypeStruct((NPAD * K, ci), f32),
        mesh=_mesh,
        scratch_types=[
            pltpu.VMEM((6 * BLK,), f32),
            pltpu.VMEM((6 * BLK,), f32),
            sv_t,
            sv_t,
            fb_t,
            fb_t,
            pltpu.VMEM((CHUNK16, ci), f32),
            pltpu.VMEM((48,), i32),
            pltpu.VMEM((NCHUNK + 16,), i32),
            pltpu.SemaphoreType.DMA,
            pltpu.SemaphoreType.DMA,
        ],
    )


_build_h16 = _make_build_h(16, False)
_build_h64 = _make_build_h(64, True)
_build_h128 = _make_build_h(128, True)


def _tc_bn(vel128, g128, b128):
    def body(v_ref, g_ref, b_ref, o_ref):
        v = v_ref[...]
        s1 = jnp.sum(v, axis=0, keepdims=True) * (1.0 / N)
        s2 = jnp.sum(v * v, axis=0, keepdims=True) * (1.0 / N)
        var = s2 - s1 * s1
        o_ref[...] = (v - s1) * lax.rsqrt(var + 1e-5) * g_ref[...] + b_ref[...]
    return pl.pallas_call(
        body, out_shape=jax.ShapeDtypeStruct((NPAD, 128), f32),
    )(vel128, g128, b128)


def _tc_layer(h2, feats, A, D, bias, res=None, WG=None, act=None,
              relu_feats=True):
    kci = A.shape[0]
    co = A.shape[1]
    ci = feats.shape[1]
    has_res = res is not None
    has_g = WG is not None

    def body(h_ref, f_ref, A_ref, D_ref, b_ref, *rest):
        rest = list(rest)
        res_ref = rest.pop(0) if has_res else None
        wg_ref = rest.pop(0) if has_g else None
        o_ref = rest.pop(0)
        g_ref = rest.pop(0) if has_g else None
        f = f_ref[...]
        if relu_feats:
            f = jnp.maximum(f, 0.0)
        x = (jnp.dot(h_ref[...], A_ref[...], preferred_element_type=f32)
             + jnp.dot(f, D_ref[...], preferred_element_type=f32)
             + b_ref[...])
        if has_res:
            x = x + res_ref[...]
        if act == "tanh":
            x = jnp.tanh(x) * 0.8 + 1.0
        o_ref[...] = x
        if has_g:
            g_ref[...] = jnp.dot(jnp.maximum(x, 0.0), wg_ref[...],
                                 preferred_element_type=f32)

    in_arrays = [h2, feats, A, D, bias]
    in_specs = [
        pl.BlockSpec((RBLK, kci), lambda i: (i, 0)),
        pl.BlockSpec((RBLK, ci), lambda i: (i, 0)),
        pl.BlockSpec((kci, co), lambda i: (0, 0)),
        pl.BlockSpec((ci, co), lambda i: (0, 0)),
        pl.BlockSpec((1, co), lambda i: (0, 0)),
    ]
    if has_res:
        in_arrays.append(res)
        in_specs.append(pl.BlockSpec((RBLK, co), lambda i: (i, 0)))
    if has_g:
        cg = WG.shape[1]
        in_arrays.append(WG)
        in_specs.append(pl.BlockSpec((co, cg), lambda i: (0, 0)))
    out_shape = [jax.ShapeDtypeStruct((NPAD, co), f32)]
    out_specs = [pl.BlockSpec((RBLK, co), lambda i: (i, 0))]
    if has_g:
        out_shape.append(jax.ShapeDtypeStruct((NPAD, cg), f32))
        out_specs.append(pl.BlockSpec((RBLK, cg), lambda i: (i, 0)))
    outs = pl.pallas_call(
        body, grid=(NPAD // RBLK,), in_specs=in_specs,
        out_specs=out_specs, out_shape=out_shape,
    )(*in_arrays)
    return outs


_WDIAG = np.zeros((256, 128), np.float32)
for _k in range(16):
    _WDIAG[_k * 16 + _k, 0] = 1.0


def kernel(pos, vel, edge_src, edge_dst, bn_gamma, bn_beta,
           conv0_W, conv0_b, dense0_W, dense0_b,
           conv1_W, conv1_b, dense1_W, dense1_b,
           conv2_W, conv2_b, dense2_W, dense2_b,
           conv3_W, conv3_b, dense3_W, dense3_b,
           conv4_W, conv4_b, dense4_W, dense4_b):
    E = edge_src.shape[0]
    src = edge_src.astype(i32)
    dst = edge_dst.astype(i32)

    chunk_of_edge = dst // CHUNK
    cnt = jnp.bincount(chunk_of_edge, length=NCHUNK).astype(i32)
    capblk = (cnt + (BLK - 1)) // BLK
    blkb = jnp.concatenate([jnp.zeros((1,), i32),
                            jnp.cumsum(capblk).astype(i32)])
    estart = jnp.concatenate([jnp.zeros((1,), i32),
                              jnp.cumsum(cnt).astype(i32)])
    slot = (blkb[chunk_of_edge] * BLK
            + (jnp.arange(E, dtype=i32) - estart[chunk_of_edge]))
    blk_ids = jnp.arange(NBLK_CAP, dtype=i32)
    chunk_of_blk = jnp.clip(
        jnp.searchsorted(blkb, blk_ids, side="right").astype(i32) - 1,
        0, NCHUNK - 1)
    n0blk = (chunk_of_blk * CHUNK).astype(i32)
    n0slot = jnp.repeat(n0blk, BLK)
    srcf = jnp.zeros((EPAD_CAP,), f32).at[slot].set(src.astype(f32))
    dstf = n0slot.astype(f32).at[slot].set(dst.astype(f32))
    validf = jnp.zeros((EPAD_CAP,), f32).at[slot].set(1.0)
    ein = jnp.concatenate([
        srcf.reshape(NBLK_CAP, BLK),
        dstf.reshape(NBLK_CAP, BLK),
        validf.reshape(NBLK_CAP, BLK),
        jnp.broadcast_to(n0blk.astype(f32)[:, None], (NBLK_CAP, 16)),
        jnp.zeros((NBLK_CAP, BLK - 16), f32),
    ], axis=1)
    nbtot = blkb[NCHUNK]
    targets = (jnp.arange(NWORK + 1, dtype=i32) * nbtot) // NWORK
    wctrl = jnp.searchsorted(blkb, targets, side="left").astype(i32)
    wctrl = wctrl.at[NWORK].set(NCHUNK)
    wctrl_pad = jnp.zeros((48,), i32).at[:NWORK + 1].set(wctrl)
    blkb_pad = jnp.zeros((NCHUNK + 16,), i32).at[:NCHUNK + 1].set(blkb)

    posx = jnp.zeros((NPAD,), f32).at[:N].set(pos[:, 0])
    posy = jnp.zeros((NPAD,), f32).at[:N].set(pos[:, 1])
    mout = _sc_meta(posx, posy, ein, blkb_pad)

    vel128 = jnp.zeros((NPAD, 128), f32).at[:N, :2].set(vel)
    g128 = jnp.ones((128,), f32).at[:3].set(bn_gamma).reshape(1, 128)
    b128 = jnp.zeros((128,), f32).at[:3].set(bn_beta).reshape(1, 128)
    fl128 = _tc_bn(vel128, g128, b128)

    h0 = _build_h16(fl128, mout, wctrl_pad, blkb_pad)
    Wf0 = conv0_W.reshape(K, 3, 32)
    Wf0p = jnp.zeros((K, 16, 32), f32).at[:, :3].set(Wf0).reshape(256, 32)
    A0 = jnp.zeros((256, 128), f32).at[:, :32].set(Wf0p)
    D0 = jnp.zeros((128, 128), f32).at[:3, 32:64].set(dense0_W)
    bias0 = jnp.zeros((128,), f32).at[:32].set(conv0_b).at[32:64].set(
        dense0_b).reshape(1, 128)
    (x0,) = _tc_layer(h0.reshape(NPAD, 256), fl128, A0, D0, bias0,
                      relu_feats=False)

    h1 = _build_h64(x0, mout, wctrl_pad, blkb_pad)
    A1 = jnp.zeros((1024, 128), f32).at[:, :64].set(
        conv1_W.reshape(1024, 64))
    D1 = jnp.zeros((128, 128), f32).at[:64, :64].set(dense1_W)
    bias1 = jnp.zeros((128,), f32).at[:64].set(conv1_b + dense1_b).reshape(1, 128)
    (x1,) = _tc_layer(h1.reshape(NPAD, 1024), x0, A1, D1, bias1,
                      res=x0, relu_feats=True)

    h2 = _build_h64(x1, mout, wctrl_pad, blkb_pad)
    A2 = conv2_W.reshape(K, 64, 128).reshape(1024, 128)
    D2 = jnp.zeros((128, 128), f32).at[:64, :].set(dense2_W)
    bias2 = (conv2_b + dense2_b).reshape(1, 128)
    (x2,) = _tc_layer(h2.reshape(NPAD, 1024), x1, A2, D2, bias2,
                      relu_feats=True)

    h3 = _build_h128(x2, mout, wctrl_pad, blkb_pad)
    A3 = conv3_W.reshape(K, 128, 256).reshape(2048, 256)
    bias3 = (conv3_b + dense3_b).reshape(1, 256)
    WG = jnp.zeros((256, 128), f32).at[:, :K].set(
        conv4_W.reshape(K, 256).T)
    x3, G4 = _tc_layer(h3.reshape(NPAD, 2048), x2, A3, dense3_W, bias3,
                       WG=WG, relu_feats=True)

    h4 = _build_h16(G4, mout, wctrl_pad, blkb_pad)
    Wdiag = jnp.asarray(_WDIAG)
    D4 = jnp.zeros((256, 128), f32).at[:, :1].set(dense4_W)
    bias4 = jnp.zeros((1, 128), f32).at[0, 0].set(conv4_b[0] + dense4_b[0])
    (x4,) = _tc_layer(h4.reshape(NPAD, 256), x3, Wdiag, D4, bias4,
                      act="tanh", relu_feats=True)
    return x4[:N, :1]


# speedup vs baseline: 3.0766x; 1.5486x over previous
import functools

import numpy as np
import jax
import jax.numpy as jnp
from jax import lax
from jax.experimental import pallas as pl
from jax.experimental.pallas import tpu as pltpu
from jax.experimental.pallas import tpu_sc as plsc

N = 10000
NPAD = 10240
RADIUS = 0.125
K = 16
CHUNK = 32
CHUNK16 = CHUNK * K
NCHUNK = NPAD // CHUNK
BLK = 128
EPAD_CAP = 208000
NBLK_CAP = EPAD_CAP // BLK
NBLK_PAD = 1648
NWORK = 32
RBLK = 256
f32 = jnp.float32
i32 = jnp.int32

_mesh = plsc.VectorSubcoreMesh(core_axis_name="c", subcore_axis_name="s")


def _hypot16(ax, ay):
    m = jnp.maximum(ax, ay)
    msafe = jnp.maximum(m, 1e-12)
    t = jnp.minimum(ax, ay) / msafe
    y = 1.0 + t * t
    r = (0.14632082 * y - 0.72323499) * y + 1.57186441
    r = r * (1.5 - 0.5 * y * r * r)
    r = r * (1.5 - 0.5 * y * r * r)
    return m * (y * r)


def _atan16(t):
    t2 = t * t
    p = jnp.full((16,), -0.0040540580, f32)
    for c in (0.0218612288, -0.0559098861, 0.0964200441, -0.1390853351,
              0.1994653599, -0.3332985605, 0.9999993329):
        p = p * t2 + c
    return t * p


def _meta_body(posx, posy, ein, blkb, mout,
               erowA, erowB, svA, svB, dvA, dvB,
               psxA, psyA, pdxA, pdyA, psxB, psyB, pdxB, pdyB,
               orow, blkv,
               sA0, sA1, sA2, sA3, sB0, sB1, sB2, sB3):
    wid = lax.axis_index("s") * 2 + lax.axis_index("c")
    pltpu.sync_copy(blkb, blkv)
    nbtot = blkv[pl.ds(NCHUNK, 16)][0]
    cntw = (nbtot - wid + (NWORK - 1)) // NWORK

    bufs = ((erowA, svA, dvA, psxA, psyA, pdxA, pdyA, sA0, sA1, sA2, sA3),
            (erowB, svB, dvB, psxB, psyB, pdxB, pdyB, sB0, sB1, sB2, sB3))

    def prefetch(t, p):
        erow, sv, dv, psx, psy, pdx, pdy, s0, s1, s2, s3 = bufs[p]
        b = wid + t * NWORK
        pltpu.sync_copy(ein.at[b], erow)
        for g in range(BLK // 16):
            s = g * 16
            sv[pl.ds(s, 16)] = erow[pl.ds(s, 16)].astype(i32)
            dv[pl.ds(s, 16)] = erow[pl.ds(BLK + s, 16)].astype(i32)
        pltpu.async_copy(posx.at[sv], psx, s0)
        pltpu.async_copy(posy.at[sv], psy, s1)
        pltpu.async_copy(posx.at[dv], pdx, s2)
        pltpu.async_copy(posy.at[dv], pdy, s3)

    def process(t, p):
        erow, sv, dv, psx, psy, pdx, pdy, s0, s1, s2, s3 = bufs[p]
        b = wid + t * NWORK
        pltpu.make_async_copy(posx.at[sv], psx, s0).wait()
        pltpu.make_async_copy(posy.at[sv], psy, s1).wait()
        pltpu.make_async_copy(posx.at[dv], pdx, s2).wait()
        pltpu.make_async_copy(posy.at[dv], pdy, s3).wait()
        n0f = erow[pl.ds(3 * BLK, 16)]

        def grp(g, c4):
            s = g * 16
            sx = psx[pl.ds(s, 16)]
            sy = psy[pl.ds(s, 16)]
            dx = pdx[pl.ds(s, 16)]
            dy = pdy[pl.ds(s, 16)]
            dlf = erow[pl.ds(BLK + s, 16)]
            vl = erow[pl.ds(2 * BLK + s, 16)]
            rx = (sx - dx) * (1.0 / RADIUS)
            ry = (sy - dy) * (1.0 / RADIUS)
            sq = rx * rx + ry * ry
            om = 1.0 - sq
            win = jnp.clip(om * om * om, 0.0, 1.0)
            ax = jnp.abs(rx)
            ay = jnp.abs(ry)
            nxy = _hypot16(ax, ay)
            condx = ax >= ay
            xs = jnp.where(ax > 1e-8, rx, 1.0)
            ys = jnp.where(ay > 1e-8, ry, 1.0)
            FOUR_PI = 1.2732395447351628
            a1 = _atan16(jnp.clip(ry / xs, -1.0, 1.0))
            u1 = jnp.sign(rx) * nxy
            v1 = u1 * FOUR_PI * a1
            a2 = _atan16(jnp.clip(rx / ys, -1.0, 1.0))
            v2 = jnp.sign(ry) * nxy
            u2 = v2 * FOUR_PI * a2
            u = jnp.where(condx, u1, u2)
            v = jnp.where(condx, v1, v2)
            tiny = sq < 1e-12
            u = jnp.where(tiny, 0.0, u)
            v = jnp.where(tiny, 0.0, v)
            gx = jnp.clip((u + 1.0) * 1.5, 0.0, 3.0)
            gy = jnp.clip((v + 1.0) * 1.5, 0.0, 3.0)
            x0i = jnp.minimum(gx.astype(i32), 2)
            y0i = jnp.minimum(gy.astype(i32), 2)
            wx1 = gx - x0i.astype(f32)
            wy1 = gy - y0i.astype(f32)
            wx0 = 1.0 - wx1
            wy0 = 1.0 - wy1
            wv_ = win * vl
            cellf = (x0i * 4 + y0i).astype(f32)
            orow[pl.ds(s, 16)] = (dlf - n0f) * float(K) + cellf
            orow[pl.ds(BLK + s, 16)] = wx0 * wy0 * wv_
            orow[pl.ds(2 * BLK + s, 16)] = wx0 * wy1 * wv_
            orow[pl.ds(3 * BLK + s, 16)] = wx1 * wy0 * wv_
            orow[pl.ds(4 * BLK + s, 16)] = wx1 * wy1 * wv_
            orow[pl.ds(5 * BLK + s, 16)] = erow[pl.ds(s, 16)]
            return c4

        lax.fori_loop(0, BLK // 16, grp, 0)
        pltpu.sync_copy(orow, mout.at[b])

    @pl.when(cntw > 0)
    def _():
        prefetch(0, 0)

    def pair_body(pair, carry):
        tA = 2 * pair
        tB = tA + 1

        @pl.when(tB < cntw)
        def _():
            prefetch(tB, 1)

        @pl.when(tA < cntw)
        def _():
            process(tA, 0)

        @pl.when(tB + 1 < cntw)
        def _():
            prefetch(tB + 1, 0)

        @pl.when(tB < cntw)
        def _():
            process(tB, 1)
        return carry

    lax.fori_loop(0, (cntw + 1) // 2, pair_body, 0)


_sc_meta = pl.kernel(
    _meta_body,
    out_type=jax.ShapeDtypeStruct((NBLK_CAP, 6 * BLK), f32),
    mesh=_mesh,
    scratch_types=[
        pltpu.VMEM((4 * BLK,), f32),
        pltpu.VMEM((4 * BLK,), f32),
        pltpu.VMEM((BLK,), i32),
        pltpu.VMEM((BLK,), i32),
        pltpu.VMEM((BLK,), i32),
        pltpu.VMEM((BLK,), i32),
        pltpu.VMEM((BLK,), f32),
        pltpu.VMEM((BLK,), f32),
        pltpu.VMEM((BLK,), f32),
        pltpu.VMEM((BLK,), f32),
        pltpu.VMEM((BLK,), f32),
        pltpu.VMEM((BLK,), f32),
        pltpu.VMEM((BLK,), f32),
        pltpu.VMEM((BLK,), f32),
        pltpu.VMEM((6 * BLK,), f32),
        pltpu.VMEM((NCHUNK + 16,), i32),
        pltpu.SemaphoreType.DMA,
        pltpu.SemaphoreType.DMA,
        pltpu.SemaphoreType.DMA,
        pltpu.SemaphoreType.DMA,
        pltpu.SemaphoreType.DMA,
        pltpu.SemaphoreType.DMA,
        pltpu.SemaphoreType.DMA,
        pltpu.SemaphoreType.DMA,
    ],
)


def _make_build_h(ci, relu):
    qn = ci // 16
    epr = 128 // ci if ci < 128 else 1
    rpg = (16 * ci + 127) // 128
    nrow = BLK * ci // 128 if ci < 128 else BLK

    def body(feats, mout, wctrl, blkb, h_out,
             mrowA, mrowB, svA, svB, fbufA, fbufB, hloc, ctrlv, blkv,
             semA, semB):
        wid = lax.axis_index("s") * 2 + lax.axis_index("c")
        pltpu.sync_copy(wctrl, ctrlv)
        pltpu.sync_copy(blkb, blkv)
        cv = ctrlv[pl.ds(wid, 16)]
        c_lo = cv[0]
        c_hi = cv[1]

        iota16 = jnp.arange(16, dtype=i32)
        bufs = ((mrowA, svA, fbufA, semA), (mrowB, svB, fbufB, semB))

        def issue_gather(p):
            mrow, sv, fbuf, sem = bufs[p]
            pltpu.async_copy(feats.at[sv], fbuf, sem)

        def drain_gather(p):
            mrow, sv, fbuf, sem = bufs[p]
            pltpu.make_async_copy(feats.at[sv], fbuf, sem).wait()

        def prefetch(b, p):
            mrow, sv, fbuf, sem = bufs[p]
            pltpu.sync_copy(mout.at[b], mrow)
            for g in range(BLK // 16):
                s = g * 16
                sv[pl.ds(s, 16)] = mrow[pl.ds(5 * BLK + s, 16)].astype(i32)
            issue_gather(p)

        def process(p):
            mrow, sv, fbuf, sem = bufs[p]
            drain_gather(p)

            def grp(g, c4):
                s = g * 16
                rv = mrow[pl.ds(s, 16)].astype(i32)
                w0v = mrow[pl.ds(BLK + s, 16)]
                w1v = mrow[pl.ds(2 * BLK + s, 16)]
                w2v = mrow[pl.ds(3 * BLK + s, 16)]
                w3v = mrow[pl.ds(4 * BLK + s, 16)]
                for l in range(16):
                    r = rv[l]
                    for q in range(qn):
                        sl = pl.ds(q * 16, 16)
                        f = fbuf[s + l, sl]
                        if relu:
                            f = jnp.maximum(f, 0.0)
                        plsc.addupdate(hloc.at[r, sl], w0v[l] * f)
                        plsc.addupdate(hloc.at[r + 1, sl], w1v[l] * f)
                        plsc.addupdate(hloc.at[r + 4, sl], w2v[l] * f)
                        plsc.addupdate(hloc.at[r + 5, sl], w3v[l] * f)
                return c4
            lax.fori_loop(0, BLK // 16, grp, 0)

        def chunk_body(j, carry):
            def zrow(r, c2):
                for q in range(qn):
                    hloc[r, pl.ds(q * 16, 16)] = jnp.zeros((16,), f32)
                return c2
            lax.fori_loop(0, CHUNK16, zrow, 0)
            bv = blkv[pl.ds(j, 16)]
            b0 = bv[0]
            nb = bv[1] - b0

            @pl.when(nb > 0)
            def _():
                prefetch(b0, 0)

            def pair_body(pair, c3):
                tA = 2 * pair
                tB = tA + 1

                @pl.when(tB < nb)
                def _():
                    prefetch(b0 + tB, 1)

                @pl.when(tA < nb)
                def _():
                    process(0)

                @pl.when(tB + 1 < nb)
                def _():
                    prefetch(b0 + tB + 1, 0)

                @pl.when(tB < nb)
                def _():
                    process(1)
                return c3

            lax.fori_loop(0, (nb + 1) // 2, pair_body, 0)
            pltpu.sync_copy(hloc, h_out.at[pl.ds(j * CHUNK16, CHUNK16)])
            return carry
        lax.fori_loop(c_lo, c_hi, chunk_body, 0)

    sv_t = pltpu.VMEM((BLK,), i32)
    fb_t = pltpu.VMEM((BLK, 128), f32)
    return pl.kernel(
        body,
        out_type=jax.ShapeDtypeStruct((NPAD * K, ci), f32),
        mesh=_mesh,
        scratch_types=[
            pltpu.VMEM((6 * BLK,), f32),
            pltpu.VMEM((6 * BLK,), f32),
            sv_t,
            sv_t,
            fb_t,
            fb_t,
            pltpu.VMEM((CHUNK16, ci), f32),
            pltpu.VMEM((48,), i32),
            pltpu.VMEM((NCHUNK + 16,), i32),
            pltpu.SemaphoreType.DMA,
            pltpu.SemaphoreType.DMA,
        ],
    )


_build_h16 = _make_build_h(16, False)
_build_h64 = _make_build_h(64, True)
_build_h128 = _make_build_h(128, True)


def _tc_bn(vel128, g128, b128):
    def body(v_ref, g_ref, b_ref, o_ref):
        v = v_ref[...]
        s1 = jnp.sum(v, axis=0, keepdims=True) * (1.0 / N)
        s2 = jnp.sum(v * v, axis=0, keepdims=True) * (1.0 / N)
        var = s2 - s1 * s1
        o_ref[...] = (v - s1) * lax.rsqrt(var + 1e-5) * g_ref[...] + b_ref[...]
    return pl.pallas_call(
        body, out_shape=jax.ShapeDtypeStruct((NPAD, 128), f32),
    )(vel128, g128, b128)


def _tc_layer(h2, feats, A, D, bias, res=None, WG=None, act=None,
              relu_feats=True):
    kci = A.shape[0]
    co = A.shape[1]
    ci = feats.shape[1]
    has_res = res is not None
    has_g = WG is not None

    def body(h_ref, f_ref, A_ref, D_ref, b_ref, *rest):
        rest = list(rest)
        res_ref = rest.pop(0) if has_res else None
        wg_ref = rest.pop(0) if has_g else None
        o_ref = rest.pop(0)
        g_ref = rest.pop(0) if has_g else None
        f = f_ref[...]
        if relu_feats:
            f = jnp.maximum(f, 0.0)
        x = (jnp.dot(h_ref[...], A_ref[...], preferred_element_type=f32)
             + jnp.dot(f, D_ref[...], preferred_element_type=f32)
             + b_ref[...])
        if has_res:
            x = x + res_ref[...]
        if act == "tanh":
            x = jnp.tanh(x) * 0.8 + 1.0
        o_ref[...] = x
        if has_g:
            g_ref[...] = jnp.dot(jnp.maximum(x, 0.0), wg_ref[...],
                                 preferred_element_type=f32)

    in_arrays = [h2, feats, A, D, bias]
    in_specs = [
        pl.BlockSpec((RBLK, kci), lambda i: (i, 0)),
        pl.BlockSpec((RBLK, ci), lambda i: (i, 0)),
        pl.BlockSpec((kci, co), lambda i: (0, 0)),
        pl.BlockSpec((ci, co), lambda i: (0, 0)),
        pl.BlockSpec((1, co), lambda i: (0, 0)),
    ]
    if has_res:
        in_arrays.append(res)
        in_specs.append(pl.BlockSpec((RBLK, co), lambda i: (i, 0)))
    if has_g:
        cg = WG.shape[1]
        in_arrays.append(WG)
        in_specs.append(pl.BlockSpec((co, cg), lambda i: (0, 0)))
    out_shape = [jax.ShapeDtypeStruct((NPAD, co), f32)]
    out_specs = [pl.BlockSpec((RBLK, co), lambda i: (i, 0))]
    if has_g:
        out_shape.append(jax.ShapeDtypeStruct((NPAD, cg), f32))
        out_specs.append(pl.BlockSpec((RBLK, cg), lambda i: (i, 0)))
    outs = pl.pallas_call(
        body, grid=(NPAD // RBLK,), in_specs=in_specs,
        out_specs=out_specs, out_shape=out_shape,
    )(*in_arrays)
    return outs


_WDIAG = np.zeros((256, 128), np.float32)
for _k in range(16):
    _WDIAG[_k * 16 + _k, 0] = 1.0


def kernel(pos, vel, edge_src, edge_dst, bn_gamma, bn_beta,
           conv0_W, conv0_b, dense0_W, dense0_b,
           conv1_W, conv1_b, dense1_W, dense1_b,
           conv2_W, conv2_b, dense2_W, dense2_b,
           conv3_W, conv3_b, dense3_W, dense3_b,
           conv4_W, conv4_b, dense4_W, dense4_b):
    E = edge_src.shape[0]
    src = edge_src.astype(i32)
    dst = edge_dst.astype(i32)

    chunk_of_edge = dst // CHUNK
    cnt = jnp.bincount(chunk_of_edge, length=NCHUNK).astype(i32)
    capblk = (cnt + (BLK - 1)) // BLK
    blkb = jnp.concatenate([jnp.zeros((1,), i32),
                            jnp.cumsum(capblk).astype(i32)])
    estart = jnp.concatenate([jnp.zeros((1,), i32),
                              jnp.cumsum(cnt).astype(i32)])
    blk_ids = jnp.arange(NBLK_CAP, dtype=i32)
    chunk_of_blk = jnp.clip(
        jnp.searchsorted(blkb, blk_ids, side="right").astype(i32) - 1,
        0, NCHUNK - 1)
    n0blk = (chunk_of_blk * CHUNK).astype(i32)
    e_idx = (estart[chunk_of_blk] + (blk_ids - blkb[chunk_of_blk]) * BLK
             )[:, None] + jnp.arange(BLK, dtype=i32)[None, :]
    e_end = estart[chunk_of_blk + 1][:, None]
    valid = (e_idx < e_end) & (e_idx >= 0)
    e_c = jnp.clip(e_idx, 0, E - 1)
    srcf = jnp.where(valid, src[e_c], 0).astype(f32)
    dstf = jnp.where(valid, dst[e_c], n0blk[:, None]).astype(f32)
    ein = jnp.concatenate([
        srcf,
        dstf,
        valid.astype(f32),
        jnp.broadcast_to(n0blk.astype(f32)[:, None], (NBLK_CAP, 16)),
        jnp.zeros((NBLK_CAP, BLK - 16), f32),
    ], axis=1)
    nbtot = blkb[NCHUNK]
    targets = (jnp.arange(NWORK + 1, dtype=i32) * nbtot) // NWORK
    wctrl = jnp.searchsorted(blkb, targets, side="left").astype(i32)
    wctrl = wctrl.at[NWORK].set(NCHUNK)
    wctrl_pad = jnp.zeros((48,), i32).at[:NWORK + 1].set(wctrl)
    blkb_pad = jnp.zeros((NCHUNK + 16,), i32).at[:NCHUNK + 1].set(blkb)

    posx = jnp.zeros((NPAD,), f32).at[:N].set(pos[:, 0])
    posy = jnp.zeros((NPAD,), f32).at[:N].set(pos[:, 1])
    mout = _sc_meta(posx, posy, ein, blkb_pad)

    vel128 = jnp.zeros((NPAD, 128), f32).at[:N, :2].set(vel)
    g128 = jnp.ones((128,), f32).at[:3].set(bn_gamma).reshape(1, 128)
    b128 = jnp.zeros((128,), f32).at[:3].set(bn_beta).reshape(1, 128)
    fl128 = _tc_bn(vel128, g128, b128)

    h0 = _build_h16(fl128, mout, wctrl_pad, blkb_pad)
    Wf0 = conv0_W.reshape(K, 3, 32)
    Wf0p = jnp.zeros((K, 16, 32), f32).at[:, :3].set(Wf0).reshape(256, 32)
    A0 = jnp.zeros((256, 128), f32).at[:, :32].set(Wf0p)
    D0 = jnp.zeros((128, 128), f32).at[:3, 32:64].set(dense0_W)
    bias0 = jnp.zeros((128,), f32).at[:32].set(conv0_b).at[32:64].set(
        dense0_b).reshape(1, 128)
    (x0,) = _tc_layer(h0.reshape(NPAD, 256), fl128, A0, D0, bias0,
                      relu_feats=False)

    h1 = _build_h64(x0, mout, wctrl_pad, blkb_pad)
    A1 = jnp.zeros((1024, 128), f32).at[:, :64].set(
        conv1_W.reshape(1024, 64))
    D1 = jnp.zeros((128, 128), f32).at[:64, :64].set(dense1_W)
    bias1 = jnp.zeros((128,), f32).at[:64].set(conv1_b + dense1_b).reshape(1, 128)
    (x1,) = _tc_layer(h1.reshape(NPAD, 1024), x0, A1, D1, bias1,
                      res=x0, relu_feats=True)

    h2 = _build_h64(x1, mout, wctrl_pad, blkb_pad)
    A2 = conv2_W.reshape(K, 64, 128).reshape(1024, 128)
    D2 = jnp.zeros((128, 128), f32).at[:64, :].set(dense2_W)
    bias2 = (conv2_b + dense2_b).reshape(1, 128)
    (x2,) = _tc_layer(h2.reshape(NPAD, 1024), x1, A2, D2, bias2,
                      relu_feats=True)

    h3 = _build_h128(x2, mout, wctrl_pad, blkb_pad)
    A3 = conv3_W.reshape(K, 128, 256).reshape(2048, 256)
    bias3 = (conv3_b + dense3_b).reshape(1, 256)
    WG = jnp.zeros((256, 128), f32).at[:, :K].set(
        conv4_W.reshape(K, 256).T)
    x3, G4 = _tc_layer(h3.reshape(NPAD, 2048), x2, A3, dense3_W, bias3,
                       WG=WG, relu_feats=True)

    h4 = _build_h16(G4, mout, wctrl_pad, blkb_pad)
    Wdiag = jnp.asarray(_WDIAG)
    D4 = jnp.zeros((256, 128), f32).at[:, :1].set(dense4_W)
    bias4 = jnp.zeros((1, 128), f32).at[0, 0].set(conv4_b[0] + dense4_b[0])
    (x4,) = _tc_layer(h4.reshape(NPAD, 256), x3, Wdiag, D4, bias4,
                      act="tanh", relu_feats=True)
    return x4[:N, :1]


# speedup vs baseline: 3.0820x; 1.0018x over previous
import functools

import numpy as np
import jax
import jax.numpy as jnp
from jax import lax
from jax.experimental import pallas as pl
from jax.experimental.pallas import tpu as pltpu
from jax.experimental.pallas import tpu_sc as plsc

N = 10000
NPAD = 10240
RADIUS = 0.125
K = 16
CHUNK = 32
CHUNK16 = CHUNK * K
NCHUNK = NPAD // CHUNK
BLK = 128
EPAD_CAP = 208000
NBLK_CAP = EPAD_CAP // BLK
NBLK_PAD = 1648
NWORK = 32
RBLK = 256
f32 = jnp.float32
i32 = jnp.int32

_mesh = plsc.VectorSubcoreMesh(core_axis_name="c", subcore_axis_name="s")


def _hypot16(ax, ay):
    m = jnp.maximum(ax, ay)
    msafe = jnp.maximum(m, 1e-12)
    t = jnp.minimum(ax, ay) / msafe
    y = 1.0 + t * t
    r = (0.14632082 * y - 0.72323499) * y + 1.57186441
    r = r * (1.5 - 0.5 * y * r * r)
    r = r * (1.5 - 0.5 * y * r * r)
    return m * (y * r)


def _atan16(t):
    t2 = t * t
    p = jnp.full((16,), -0.0040540580, f32)
    for c in (0.0218612288, -0.0559098861, 0.0964200441, -0.1390853351,
              0.1994653599, -0.3332985605, 0.9999993329):
        p = p * t2 + c
    return t * p


def _meta_body(posx, posy, ein, blkb, mout,
               erowA, erowB, svA, svB, dvA, dvB,
               psxA, psyA, pdxA, pdyA, psxB, psyB, pdxB, pdyB,
               orow, blkv,
               sA0, sA1, sA2, sA3, sB0, sB1, sB2, sB3):
    wid = lax.axis_index("s") * 2 + lax.axis_index("c")
    pltpu.sync_copy(blkb, blkv)
    nbtot = blkv[pl.ds(NCHUNK, 16)][0]
    cntw = (nbtot - wid + (NWORK - 1)) // NWORK

    bufs = ((erowA, svA, dvA, psxA, psyA, pdxA, pdyA, sA0, sA1, sA2, sA3),
            (erowB, svB, dvB, psxB, psyB, pdxB, pdyB, sB0, sB1, sB2, sB3))

    def prefetch(t, p):
        erow, sv, dv, psx, psy, pdx, pdy, s0, s1, s2, s3 = bufs[p]
        b = wid + t * NWORK
        pltpu.sync_copy(ein.at[b], erow)
        for g in range(BLK // 16):
            s = g * 16
            sv[pl.ds(s, 16)] = erow[pl.ds(s, 16)].astype(i32)
            dv[pl.ds(s, 16)] = erow[pl.ds(BLK + s, 16)].astype(i32)
        pltpu.async_copy(posx.at[sv], psx, s0)
        pltpu.async_copy(posy.at[sv], psy, s1)
        pltpu.async_copy(posx.at[dv], pdx, s2)
        pltpu.async_copy(posy.at[dv], pdy, s3)

    def process(t, p):
        erow, sv, dv, psx, psy, pdx, pdy, s0, s1, s2, s3 = bufs[p]
        b = wid + t * NWORK
        pltpu.make_async_copy(posx.at[sv], psx, s0).wait()
        pltpu.make_async_copy(posy.at[sv], psy, s1).wait()
        pltpu.make_async_copy(posx.at[dv], pdx, s2).wait()
        pltpu.make_async_copy(posy.at[dv], pdy, s3).wait()
        n0f = erow[pl.ds(3 * BLK, 16)]

        def grp(g, c4):
            s = g * 16
            sx = psx[pl.ds(s, 16)]
            sy = psy[pl.ds(s, 16)]
            dx = pdx[pl.ds(s, 16)]
            dy = pdy[pl.ds(s, 16)]
            dlf = erow[pl.ds(BLK + s, 16)]
            vl = erow[pl.ds(2 * BLK + s, 16)]
            rx = (sx - dx) * (1.0 / RADIUS)
            ry = (sy - dy) * (1.0 / RADIUS)
            sq = rx * rx + ry * ry
            om = 1.0 - sq
            win = jnp.clip(om * om * om, 0.0, 1.0)
            ax = jnp.abs(rx)
            ay = jnp.abs(ry)
            nxy = _hypot16(ax, ay)
            condx = ax >= ay
            xs = jnp.where(ax > 1e-8, rx, 1.0)
            ys = jnp.where(ay > 1e-8, ry, 1.0)
            FOUR_PI = 1.2732395447351628
            a1 = _atan16(jnp.clip(ry / xs, -1.0, 1.0))
            u1 = jnp.sign(rx) * nxy
            v1 = u1 * FOUR_PI * a1
            a2 = _atan16(jnp.clip(rx / ys, -1.0, 1.0))
            v2 = jnp.sign(ry) * nxy
            u2 = v2 * FOUR_PI * a2
            u = jnp.where(condx, u1, u2)
            v = jnp.where(condx, v1, v2)
            tiny = sq < 1e-12
            u = jnp.where(tiny, 0.0, u)
            v = jnp.where(tiny, 0.0, v)
            gx = jnp.clip((u + 1.0) * 1.5, 0.0, 3.0)
            gy = jnp.clip((v + 1.0) * 1.5, 0.0, 3.0)
            x0i = jnp.minimum(gx.astype(i32), 2)
            y0i = jnp.minimum(gy.astype(i32), 2)
            wx1 = gx - x0i.astype(f32)
            wy1 = gy - y0i.astype(f32)
            wx0 = 1.0 - wx1
            wy0 = 1.0 - wy1
            wv_ = win * vl
            cellf = (x0i * 4 + y0i).astype(f32)
            orow[pl.ds(s, 16)] = (dlf - n0f) * float(K) + cellf
            orow[pl.ds(BLK + s, 16)] = wx0 * wy0 * wv_
            orow[pl.ds(2 * BLK + s, 16)] = wx0 * wy1 * wv_
            orow[pl.ds(3 * BLK + s, 16)] = wx1 * wy0 * wv_
            orow[pl.ds(4 * BLK + s, 16)] = wx1 * wy1 * wv_
            orow[pl.ds(5 * BLK + s, 16)] = erow[pl.ds(s, 16)]
            return c4

        lax.fori_loop(0, BLK // 16, grp, 0)
        pltpu.sync_copy(orow, mout.at[b])

    @pl.when(cntw > 0)
    def _():
        prefetch(0, 0)

    def pair_body(pair, carry):
        tA = 2 * pair
        tB = tA + 1

        @pl.when(tB < cntw)
        def _():
            prefetch(tB, 1)

        @pl.when(tA < cntw)
        def _():
            process(tA, 0)

        @pl.when(tB + 1 < cntw)
        def _():
            prefetch(tB + 1, 0)

        @pl.when(tB < cntw)
        def _():
            process(tB, 1)
        return carry

    lax.fori_loop(0, (cntw + 1) // 2, pair_body, 0)


_sc_meta = pl.kernel(
    _meta_body,
    out_type=jax.ShapeDtypeStruct((NBLK_CAP, 6 * BLK), f32),
    mesh=_mesh,
    scratch_types=[
        pltpu.VMEM((4 * BLK,), f32),
        pltpu.VMEM((4 * BLK,), f32),
        pltpu.VMEM((BLK,), i32),
        pltpu.VMEM((BLK,), i32),
        pltpu.VMEM((BLK,), i32),
        pltpu.VMEM((BLK,), i32),
        pltpu.VMEM((BLK,), f32),
        pltpu.VMEM((BLK,), f32),
        pltpu.VMEM((BLK,), f32),
        pltpu.VMEM((BLK,), f32),
        pltpu.VMEM((BLK,), f32),
        pltpu.VMEM((BLK,), f32),
        pltpu.VMEM((BLK,), f32),
        pltpu.VMEM((BLK,), f32),
        pltpu.VMEM((6 * BLK,), f32),
        pltpu.VMEM((NCHUNK + 16,), i32),
        pltpu.SemaphoreType.DMA,
        pltpu.SemaphoreType.DMA,
        pltpu.SemaphoreType.DMA,
        pltpu.SemaphoreType.DMA,
        pltpu.SemaphoreType.DMA,
        pltpu.SemaphoreType.DMA,
        pltpu.SemaphoreType.DMA,
        pltpu.SemaphoreType.DMA,
    ],
)


def _make_build_h(ci, relu, depth=2):
    qn = ci // 16

    def body(feats, mout, wctrl, blkb, h_out, *scratch):
        mrows = scratch[0:depth]
        svs = scratch[depth:2 * depth]
        fbufs = scratch[2 * depth:3 * depth]
        hloc, ctrlv, blkv = scratch[3 * depth:3 * depth + 3]
        sems = scratch[3 * depth + 3:]
        wid = lax.axis_index("s") * 2 + lax.axis_index("c")
        pltpu.sync_copy(wctrl, ctrlv)
        pltpu.sync_copy(blkb, blkv)
        cv = ctrlv[pl.ds(wid, 16)]
        c_lo = cv[0]
        c_hi = cv[1]

        bufs = tuple((mrows[p], svs[p], fbufs[p], sems[p])
                     for p in range(depth))

        def issue_gather(p):
            mrow, sv, fbuf, sem = bufs[p]
            pltpu.async_copy(feats.at[sv], fbuf, sem)

        def drain_gather(p):
            mrow, sv, fbuf, sem = bufs[p]
            pltpu.make_async_copy(feats.at[sv], fbuf, sem).wait()

        def prefetch(b, p):
            mrow, sv, fbuf, sem = bufs[p]
            pltpu.sync_copy(mout.at[b], mrow)
            for g in range(BLK // 16):
                s = g * 16
                sv[pl.ds(s, 16)] = mrow[pl.ds(5 * BLK + s, 16)].astype(i32)
            issue_gather(p)

        def process(p):
            mrow, sv, fbuf, sem = bufs[p]
            drain_gather(p)

            def grp(g, c4):
                s = g * 16
                rv = mrow[pl.ds(s, 16)].astype(i32)
                w0v = mrow[pl.ds(BLK + s, 16)]
                w1v = mrow[pl.ds(2 * BLK + s, 16)]
                w2v = mrow[pl.ds(3 * BLK + s, 16)]
                w3v = mrow[pl.ds(4 * BLK + s, 16)]
                for l in range(16):
                    r = rv[l]
                    for q in range(qn):
                        sl = pl.ds(q * 16, 16)
                        f = fbuf[s + l, sl]
                        if relu:
                            f = jnp.maximum(f, 0.0)
                        plsc.addupdate(hloc.at[r, sl], w0v[l] * f)
                        plsc.addupdate(hloc.at[r + 1, sl], w1v[l] * f)
                        plsc.addupdate(hloc.at[r + 4, sl], w2v[l] * f)
                        plsc.addupdate(hloc.at[r + 5, sl], w3v[l] * f)
                return c4
            lax.fori_loop(0, BLK // 16, grp, 0)

        def chunk_body(j, carry):
            def zrow(r, c2):
                for q in range(qn):
                    hloc[r, pl.ds(q * 16, 16)] = jnp.zeros((16,), f32)
                return c2
            lax.fori_loop(0, CHUNK16, zrow, 0)
            bv = blkv[pl.ds(j, 16)]
            b0 = bv[0]
            nb = bv[1] - b0

            for k in range(depth):
                @pl.when(k < nb)
                def _(k=k):
                    prefetch(b0 + k, k)

            def step(t, c3):
                ph = lax.rem(t, depth)
                for k in range(depth):
                    sel = ph == k

                    @pl.when(sel)
                    def _(k=k):
                        process(k)

                    @pl.when(sel & (t + depth < nb))
                    def _(k=k):
                        prefetch(b0 + t + depth, k)
                return c3

            lax.fori_loop(0, nb, step, 0)
            pltpu.sync_copy(hloc, h_out.at[pl.ds(j * CHUNK16, CHUNK16)])
            return carry
        lax.fori_loop(c_lo, c_hi, chunk_body, 0)

    scratch = (
        [pltpu.VMEM((6 * BLK,), f32) for _ in range(depth)]
        + [pltpu.VMEM((BLK,), i32) for _ in range(depth)]
        + [pltpu.VMEM((BLK, 128), f32) for _ in range(depth)]
        + [pltpu.VMEM((CHUNK16, ci), f32),
           pltpu.VMEM((48,), i32),
           pltpu.VMEM((NCHUNK + 16,), i32)]
        + [pltpu.SemaphoreType.DMA for _ in range(depth)]
    )
    return pl.kernel(
        body,
        out_type=jax.ShapeDtypeStruct((NPAD * K, ci), f32),
        mesh=_mesh,
        scratch_types=scratch,
    )


_build_h16 = _make_build_h(16, False, depth=3)
_build_h64 = _make_build_h(64, True, depth=3)
_build_h128 = _make_build_h(128, True, depth=2)


def _tc_bn(vel128, g128, b128):
    def body(v_ref, g_ref, b_ref, o_ref):
        v = v_ref[...]
        s1 = jnp.sum(v, axis=0, keepdims=True) * (1.0 / N)
        s2 = jnp.sum(v * v, axis=0, keepdims=True) * (1.0 / N)
        var = s2 - s1 * s1
        o_ref[...] = (v - s1) * lax.rsqrt(var + 1e-5) * g_ref[...] + b_ref[...]
    return pl.pallas_call(
        body, out_shape=jax.ShapeDtypeStruct((NPAD, 128), f32),
    )(vel128, g128, b128)


def _tc_layer(h2, feats, A, D, bias, res=None, WG=None, act=None,
              relu_feats=True):
    kci = A.shape[0]
    co = A.shape[1]
    ci = feats.shape[1]
    has_res = res is not None
    has_g = WG is not None

    def body(h_ref, f_ref, A_ref, D_ref, b_ref, *rest):
        rest = list(rest)
        res_ref = rest.pop(0) if has_res else None
        wg_ref = rest.pop(0) if has_g else None
        o_ref = rest.pop(0)
        g_ref = rest.pop(0) if has_g else None
        f = f_ref[...]
        if relu_feats:
            f = jnp.maximum(f, 0.0)
        x = (jnp.dot(h_ref[...], A_ref[...], preferred_element_type=f32)
             + jnp.dot(f, D_ref[...], preferred_element_type=f32)
             + b_ref[...])
        if has_res:
            x = x + res_ref[...]
        if act == "tanh":
            x = jnp.tanh(x) * 0.8 + 1.0
        o_ref[...] = x
        if has_g:
            g_ref[...] = jnp.dot(jnp.maximum(x, 0.0), wg_ref[...],
                                 preferred_element_type=f32)

    in_arrays = [h2, feats, A, D, bias]
    in_specs = [
        pl.BlockSpec((RBLK, kci), lambda i: (i, 0)),
        pl.BlockSpec((RBLK, ci), lambda i: (i, 0)),
        pl.BlockSpec((kci, co), lambda i: (0, 0)),
        pl.BlockSpec((ci, co), lambda i: (0, 0)),
        pl.BlockSpec((1, co), lambda i: (0, 0)),
    ]
    if has_res:
        in_arrays.append(res)
        in_specs.append(pl.BlockSpec((RBLK, co), lambda i: (i, 0)))
    if has_g:
        cg = WG.shape[1]
        in_arrays.append(WG)
        in_specs.append(pl.BlockSpec((co, cg), lambda i: (0, 0)))
    out_shape = [jax.ShapeDtypeStruct((NPAD, co), f32)]
    out_specs = [pl.BlockSpec((RBLK, co), lambda i: (i, 0))]
    if has_g:
        out_shape.append(jax.ShapeDtypeStruct((NPAD, cg), f32))
        out_specs.append(pl.BlockSpec((RBLK, cg), lambda i: (i, 0)))
    outs = pl.pallas_call(
        body, grid=(NPAD // RBLK,), in_specs=in_specs,
        out_specs=out_specs, out_shape=out_shape,
    )(*in_arrays)
    return outs


_WDIAG = np.zeros((256, 128), np.float32)
for _k in range(16):
    _WDIAG[_k * 16 + _k, 0] = 1.0


def kernel(pos, vel, edge_src, edge_dst, bn_gamma, bn_beta,
           conv0_W, conv0_b, dense0_W, dense0_b,
           conv1_W, conv1_b, dense1_W, dense1_b,
           conv2_W, conv2_b, dense2_W, dense2_b,
           conv3_W, conv3_b, dense3_W, dense3_b,
           conv4_W, conv4_b, dense4_W, dense4_b):
    E = edge_src.shape[0]
    src = edge_src.astype(i32)
    dst = edge_dst.astype(i32)

    chunk_of_edge = dst // CHUNK
    cnt = jnp.bincount(chunk_of_edge, length=NCHUNK).astype(i32)
    capblk = (cnt + (BLK - 1)) // BLK
    blkb = jnp.concatenate([jnp.zeros((1,), i32),
                            jnp.cumsum(capblk).astype(i32)])
    estart = jnp.concatenate([jnp.zeros((1,), i32),
                              jnp.cumsum(cnt).astype(i32)])
    blk_ids = jnp.arange(NBLK_CAP, dtype=i32)
    chunk_of_blk = jnp.clip(
        jnp.searchsorted(blkb, blk_ids, side="right").astype(i32) - 1,
        0, NCHUNK - 1)
    n0blk = (chunk_of_blk * CHUNK).astype(i32)
    e_idx = (estart[chunk_of_blk] + (blk_ids - blkb[chunk_of_blk]) * BLK
             )[:, None] + jnp.arange(BLK, dtype=i32)[None, :]
    e_end = estart[chunk_of_blk + 1][:, None]
    valid = (e_idx < e_end) & (e_idx >= 0)
    e_c = jnp.clip(e_idx, 0, E - 1)
    srcf = jnp.where(valid, src[e_c], 0).astype(f32)
    dstf = jnp.where(valid, dst[e_c], n0blk[:, None]).astype(f32)
    ein = jnp.concatenate([
        srcf,
        dstf,
        valid.astype(f32),
        jnp.broadcast_to(n0blk.astype(f32)[:, None], (NBLK_CAP, 16)),
        jnp.zeros((NBLK_CAP, BLK - 16), f32),
    ], axis=1)
    nbtot = blkb[NCHUNK]
    targets = (jnp.arange(NWORK + 1, dtype=i32) * nbtot) // NWORK
    wctrl = jnp.searchsorted(blkb, targets, side="left").astype(i32)
    wctrl = wctrl.at[NWORK].set(NCHUNK)
    wctrl_pad = jnp.zeros((48,), i32).at[:NWORK + 1].set(wctrl)
    blkb_pad = jnp.zeros((NCHUNK + 16,), i32).at[:NCHUNK + 1].set(blkb)

    posx = jnp.zeros((NPAD,), f32).at[:N].set(pos[:, 0])
    posy = jnp.zeros((NPAD,), f32).at[:N].set(pos[:, 1])
    mout = _sc_meta(posx, posy, ein, blkb_pad)

    vel128 = jnp.zeros((NPAD, 128), f32).at[:N, :2].set(vel)
    g128 = jnp.ones((128,), f32).at[:3].set(bn_gamma).reshape(1, 128)
    b128 = jnp.zeros((128,), f32).at[:3].set(bn_beta).reshape(1, 128)
    fl128 = _tc_bn(vel128, g128, b128)

    h0 = _build_h16(fl128, mout, wctrl_pad, blkb_pad)
    Wf0 = conv0_W.reshape(K, 3, 32)
    Wf0p = jnp.zeros((K, 16, 32), f32).at[:, :3].set(Wf0).reshape(256, 32)
    A0 = jnp.zeros((256, 128), f32).at[:, :32].set(Wf0p)
    D0 = jnp.zeros((128, 128), f32).at[:3, 32:64].set(dense0_W)
    bias0 = jnp.zeros((128,), f32).at[:32].set(conv0_b).at[32:64].set(
        dense0_b).reshape(1, 128)
    (x0,) = _tc_layer(h0.reshape(NPAD, 256), fl128, A0, D0, bias0,
                      relu_feats=False)

    h1 = _build_h64(x0, mout, wctrl_pad, blkb_pad)
    A1 = jnp.zeros((1024, 128), f32).at[:, :64].set(
        conv1_W.reshape(1024, 64))
    D1 = jnp.zeros((128, 128), f32).at[:64, :64].set(dense1_W)
    bias1 = jnp.zeros((128,), f32).at[:64].set(conv1_b + dense1_b).reshape(1, 128)
    (x1,) = _tc_layer(h1.reshape(NPAD, 1024), x0, A1, D1, bias1,
                      res=x0, relu_feats=True)

    h2 = _build_h64(x1, mout, wctrl_pad, blkb_pad)
    A2 = conv2_W.reshape(K, 64, 128).reshape(1024, 128)
    D2 = jnp.zeros((128, 128), f32).at[:64, :].set(dense2_W)
    bias2 = (conv2_b + dense2_b).reshape(1, 128)
    (x2,) = _tc_layer(h2.reshape(NPAD, 1024), x1, A2, D2, bias2,
                      relu_feats=True)

    h3 = _build_h128(x2, mout, wctrl_pad, blkb_pad)
    A3 = conv3_W.reshape(K, 128, 256).reshape(2048, 256)
    bias3 = (conv3_b + dense3_b).reshape(1, 256)
    WG = jnp.zeros((256, 128), f32).at[:, :K].set(
        conv4_W.reshape(K, 256).T)
    x3, G4 = _tc_layer(h3.reshape(NPAD, 2048), x2, A3, dense3_W, bias3,
                       WG=WG, relu_feats=True)

    h4 = _build_h16(G4, mout, wctrl_pad, blkb_pad)
    Wdiag = jnp.asarray(_WDIAG)
    D4 = jnp.zeros((256, 128), f32).at[:, :1].set(dense4_W)
    bias4 = jnp.zeros((1, 128), f32).at[0, 0].set(conv4_b[0] + dense4_b[0])
    (x4,) = _tc_layer(h4.reshape(NPAD, 256), x3, Wdiag, D4, bias4,
                      act="tanh", relu_feats=True)
    return x4[:N, :1]
